# baseline (speedup 1.0000x reference)
import functools

import jax
import jax.numpy as jnp
from jax import lax
from jax.experimental import pallas as pl
from jax.experimental.pallas import tpu as pltpu
from jax.experimental.pallas import tpu_sc as plsc

NC = 2
NS = 16
NBUF = 3


@functools.partial(jax.jit, static_argnames=("batch", "num_fields", "rows", "embed_dim"))
def _fused(catr, weight, *, batch, num_fields, rows, embed_dim):
    per_table = rows // num_fields
    mesh = plsc.VectorSubcoreMesh(core_axis_name="c", subcore_axis_name="s")

    bt_per_f = batch // 128
    units = num_fields * bt_per_f
    units_sc = units // NC
    units_tile = units_sc // NS
    dsub = embed_dim // 8

    @functools.partial(
        pl.kernel,
        out_type=[jax.ShapeDtypeStruct((num_fields, dsub, bt_per_f, 8, 128),
                                       jnp.float32),
                  jax.ShapeDtypeStruct((units, 128, embed_dim), jnp.float32)],
        mesh=mesh,
        compiler_params=pltpu.CompilerParams(use_tc_tiling_on_sc=False,
                                             needs_layout_passes=False),
        scratch_types=[
            pltpu.VMEM((units_tile, 128), jnp.int32),
            pltpu.VMEM((NBUF, 128, embed_dim), jnp.float32),
            pltpu.VMEM((2 * embed_dim, 128), jnp.float32),
            pltpu.SemaphoreType.DMA((NBUF,)),
            pltpu.SemaphoreType.DMA((2,)),
        ],
    )
    def run(catr_hbm, w_hbm, out5_hbm, outflat_hbm, idx_v, rows_v, tos, gsem, wsem):
        c = lax.axis_index("c")
        s = lax.axis_index("s")
        lane = lax.iota(jnp.int32, 16)

        ubase = c * units_sc + s * units_tile
        pltpu.sync_copy(catr_hbm.at[pl.ds(ubase, units_tile)], idx_v)

        def add_body(j):
            f = (ubase + j) // bt_per_f
            off = f * per_table
            for h in range(128 // 16):
                sl = pl.ds(h * 16, 16)
                idx_v[j, sl] = idx_v[j, sl] + off

        pl.loop(0, units_tile)(add_body)

        def gather(j, b):
            jj = jnp.minimum(j, units_tile - 1)
            return (w_hbm.at[idx_v.at[jj]], rows_v.at[b], gsem.at[b])

        def unit_writes(j, q):
            u = ubase + j
            f = u // bt_per_f
            bt = u % bt_per_f
            return [(tos.at[pl.ds(q * embed_dim + dt * 8, 8)],
                     out5_hbm.at[f, dt, bt], wsem.at[q])
                    for dt in range(dsub)]

        def transpose_unit(b, q):
            def tr_body(d):
                ln = lax.iota(jnp.int32, 16)
                bv = ln * 0 + b
                dv = ln * 0 + d
                for h in range(128 // 16):
                    val = plsc.load_gather(rows_v, [bv, h * 16 + ln, dv])
                    tos[q * embed_dim + d, pl.ds(h * 16, 16)] = val
            pl.loop(0, embed_dim)(tr_body)

        for b in range(NBUF):
            pltpu.async_copy(*gather(b, b))

        def p2_body(j):
            b = j % NBUF
            q = j % 2
            pltpu.make_async_copy(*gather(j, b)).wait()
            pltpu.sync_copy(rows_v.at[b, pl.ds(0, 8)], outflat_hbm.at[c, s])

            def _wait_writes():
                for w in unit_writes(0, q):
                    pltpu.make_async_copy(*w).wait()

            pl.when(j >= 2)(_wait_writes)
            transpose_unit(b, q)
            for w in unit_writes(j, q):
                pltpu.async_copy(*w)
            pltpu.async_copy(*gather(j + NBUF, b))

        pl.loop(0, units_tile)(p2_body)
        for b in range(NBUF):
            pltpu.make_async_copy(*gather(0, b)).wait()
        for q in range(2):
            for w in unit_writes(0, q):
                pltpu.make_async_copy(*w).wait()

    return run(catr, weight)


def kernel(categorical_inputs, weight):
    B, F = categorical_inputs.shape
    R, D = weight.shape
    catr = categorical_inputs.T.reshape(F * (B // 128), 128)
    out5, outflat = _fused(catr, weight, batch=B, num_fields=F, rows=R,
                           embed_dim=D)
    del outflat
    return out5.transpose(2, 4, 0, 1, 3).reshape(B, F, D)


# speedup vs baseline: 1.1571x; 1.1571x over previous
import functools

import jax
import jax.numpy as jnp
from jax import lax
from jax.experimental import pallas as pl
from jax.experimental.pallas import tpu as pltpu
from jax.experimental.pallas import tpu_sc as plsc

NC = 2
NS = 16
NW = NC * NS
CHUNK = 128
NBUF = 8


@functools.partial(jax.jit, static_argnames=("total", "embed_dim", "j_per_w", "num_fields"))
def _fused_gather(cat3, weight, *, total, embed_dim, j_per_w, num_fields):
    b_per_w = j_per_w * CHUNK
    per_table = weight.shape[0] // num_fields
    mesh = plsc.VectorSubcoreMesh(core_axis_name="c", subcore_axis_name="s")
    n_rounds = j_per_w // NBUF

    @functools.partial(
        pl.kernel,
        out_type=jax.ShapeDtypeStruct((total, embed_dim), jnp.float32),
        mesh=mesh,
        compiler_params=pltpu.CompilerParams(use_tc_tiling_on_sc=False),
        scratch_types=[
            pltpu.VMEM((j_per_w, CHUNK), jnp.int32),
            pltpu.VMEM((NBUF, CHUNK, embed_dim), jnp.float32),
            pltpu.SemaphoreType.DMA((NBUF,)),
            pltpu.SemaphoreType.DMA((NBUF,)),
        ],
    )
    def run(cat_hbm, w_hbm, out_hbm, idx_v, rows_v, gsem, wsem):
        wid = lax.axis_index("s") * NC + lax.axis_index("c")
        pltpu.sync_copy(cat_hbm.at[wid], idx_v)

        lane = lax.iota(jnp.int32, 16)
        steps_per_row = CHUNK // 16

        def add_body(t, f):
            j = t // steps_per_row
            i = (t % steps_per_row) * 16
            idx_v[j, pl.ds(i, 16)] = idx_v[j, pl.ds(i, 16)] + f * per_table
            fn = f + 16
            return jnp.where(fn >= num_fields, fn - num_fields, fn)

        lax.fori_loop(0, j_per_w * steps_per_row, add_body,
                      lane % num_fields)

        base = wid * b_per_w

        def gather_args(j, b):
            return (w_hbm.at[idx_v.at[j]], rows_v.at[b], gsem.at[b])

        def write_args(j, b):
            return (rows_v.at[b],
                    out_hbm.at[pl.ds(base + j * CHUNK, CHUNK)], wsem.at[b])

        for b in range(NBUF):
            pltpu.async_copy(*gather_args(b, b))

        def round_body(g, refill):
            j0 = g * NBUF
            for b in range(NBUF):
                pltpu.make_async_copy(*gather_args(j0 + b, b)).wait()
                pltpu.async_copy(*write_args(j0 + b, b))
            for b in range(NBUF):
                pltpu.make_async_copy(*write_args(j0 + b, b)).wait()
                if refill:
                    pltpu.async_copy(*gather_args(j0 + NBUF + b, b))

        pl.loop(0, n_rounds - 1)(lambda g: round_body(g, True))
        round_body(n_rounds - 1, False)

    return run(cat3, weight)


def kernel(categorical_inputs, weight):
    B, F = categorical_inputs.shape
    R, D = weight.shape
    total = B * F
    assert total % (NW * CHUNK) == 0
    j_per_w = total // (NW * CHUNK)
    assert j_per_w % NBUF == 0

    cat3 = categorical_inputs.reshape(NW, j_per_w, CHUNK)
    out = _fused_gather(cat3, weight, total=total, embed_dim=D,
                        j_per_w=j_per_w, num_fields=F)
    return out.reshape(B, F, D)
